# baseline (speedup 1.0000x reference)
import functools
import jax
import jax.numpy as jnp
from jax import lax
from jax.experimental import pallas as pl
from jax.experimental.pallas import tpu as pltpu
from jax.experimental.pallas import tpu_sc as plsc

D = 64
NC, NS = 2, 16
NW = NC * NS
CB = 4
SPLITS = ((0, 104), (104, 96))


def _make_gather(B, L):
    per_w = B // NW
    n_chunks = per_w // CB
    n_pairs = n_chunks // 2
    chunk_rows = CB * L

    @functools.partial(
        pl.kernel,
        mesh=plsc.VectorSubcoreMesh(core_axis_name="c", subcore_axis_name="s"),
        out_type=jax.ShapeDtypeStruct((B, L, D), jnp.float32),
        scratch_types=[
            pltpu.VMEM((2, CB, L), jnp.int32),
            pltpu.VMEM((CB, L, D), jnp.float32),
            pltpu.VMEM((CB, L, D), jnp.float32),
            pltpu.SemaphoreType.DMA,
            pltpu.SemaphoreType.DMA,
            pltpu.SemaphoreType.DMA,
            pltpu.SemaphoreType.DMA,
        ],
        compiler_params=pltpu.CompilerParams(use_tc_tiling_on_sc=False),
    )
    def gather_kernel(idx_hbm, table_hbm, out_hbm, idx_v,
                      rows0, rows1, sg0, sg1, so0, so1):
        rows = [rows0, rows1]
        sg = [sg0, sg1]
        so = [so0, so1]
        wid = lax.axis_index("s") * NC + lax.axis_index("c")
        batch0 = wid * per_w

        def fire_gather(ci, p):
            b0 = batch0 + ci * CB
            pltpu.sync_copy(idx_hbm.at[pl.ds(b0, CB)], idx_v.at[p])
            for r in range(CB):
                for off, ln in SPLITS:
                    pltpu.async_copy(
                        table_hbm.at[idx_v.at[p, r, pl.ds(off, ln)]],
                        rows[p].at[r, pl.ds(off, ln)],
                        sg[p],
                    )

        def drain_gather(p):
            pltpu.make_async_copy(
                out_hbm.at[pl.ds(0, CB)], rows[p], sg[p]
            ).wait()

        def fire_out(ci, p):
            b0 = batch0 + ci * CB
            pltpu.async_copy(rows[p], out_hbm.at[pl.ds(b0, CB)], so[p])

        def drain_out(p):
            pltpu.make_async_copy(
                rows[p], out_hbm.at[pl.ds(0, CB)], so[p]
            ).wait()

        fire_gather(0, 0)

        def body(m, carry):
            ci = 2 * m

            @pl.when(m > 0)
            def _():
                drain_out(1)

            fire_gather(ci + 1, 1)
            drain_gather(0)
            fire_out(ci, 0)
            drain_out(0)

            @pl.when(m < n_pairs - 1)
            def _():
                fire_gather(ci + 2, 0)

            drain_gather(1)
            fire_out(ci + 1, 1)
            return carry

        lax.fori_loop(0, n_pairs, body, 0)
        drain_out(1)

    return gather_kernel


_gather = _make_gather(4096, 200)


def kernel(batch_data, table):
    return _gather(batch_data.astype(jnp.int32), table)
